# baseline (speedup 1.0000x reference)
import functools

import jax
import jax.numpy as jnp
from jax import lax
from jax.experimental import pallas as pl
from jax.experimental.pallas import tpu as pltpu
from jax.experimental.pallas import tpu_sc as plsc

G = 100000
D = 32
B = 16384
EPS = 1e-7

NC = 2
NS = 16
L = 16
NW = NC * NS
BPW = B // NW
CHUNK = 128
NCHUNK = BPW // CHUNK

Q = 25600
RROWS = 2 * Q

BLKC = 5120
TCGRID = Q // BLKC


def _loc_of(x):
    return EPS + 1.0 / (1.0 + jnp.exp(-x))


def _scale_of(x):
    return EPS + jnp.maximum(x, 0.0) + jnp.log(1.0 + jnp.exp(-jnp.abs(x)))


def _pack16(x):
    b = lax.bitcast_convert_type(
        x.astype(jnp.bfloat16), jnp.uint16).astype(jnp.uint32)
    w = b[0:16, :] | (b[16:32, :] << 16)
    return lax.bitcast_convert_type(w, jnp.float32)


def _tc_prep_body(ul0, us0, ul1, us1, ul2, us2, ul3, us3, c_ref):
    srcs = []
    for ulr, usr in ((ul0, us0), (ul1, us1), (ul2, us2), (ul3, us3)):
        srcs.append(_pack16(_loc_of(ulr[...])))
        srcs.append(_pack16(_scale_of(usr[...])))
    for c in range(BLKC // 128):
        lo, hi = c * 128, (c + 1) * 128
        sq = jnp.concatenate([s[:, lo:hi] for s in srcs], axis=0)
        c_ref[pl.ds(c * 128 * 128, 128 * 128)] = sq.T.reshape(128 * 128)


def _in_spec(r):
    return pl.BlockSpec((32, BLKC), lambda i, _r=r: (0, i + _r * TCGRID))


_tc_prep = pl.pallas_call(
    _tc_prep_body,
    grid=(TCGRID,),
    in_specs=[_in_spec(r) for r in (0, 0, 1, 1, 2, 2, 3, 3)],
    out_specs=pl.BlockSpec((BLKC * 128,), lambda i: (i,)),
    out_shape=jax.ShapeDtypeStruct((RROWS * 64,), jnp.float32),
)


NZBLK = 2048


def _nz_prep_body(n0, n1, n2, n3, it_ref, nz_ref, idx_ref):
    qs = [n0[...], n1[...], n2[...], n3[...]]
    for c in range(NZBLK // 128):
        lo, hi = c * 128, (c + 1) * 128
        sq = jnp.concatenate([q[:, lo:hi] for q in qs], axis=0)
        nz_ref[lo:hi, :] = sq.T
    idx_ref[...] = it_ref[...].reshape(B // 2)


_nz_prep = pl.pallas_call(
    _nz_prep_body,
    grid=(2,),
    in_specs=[
        pl.BlockSpec((32, NZBLK), lambda i: (0, i)),
        pl.BlockSpec((32, NZBLK), lambda i: (0, i + 2)),
        pl.BlockSpec((32, NZBLK), lambda i: (0, i + 4)),
        pl.BlockSpec((32, NZBLK), lambda i: (0, i + 6)),
        pl.BlockSpec((1, B // 2), lambda i: (0, i)),
    ],
    out_specs=[
        pl.BlockSpec((NZBLK, 128), lambda i: (i, 0)),
        pl.BlockSpec((B // 2,), lambda i: (i,)),
    ],
    out_shape=[
        jax.ShapeDtypeStruct((B // 4, 128), jnp.float32),
        jax.ShapeDtypeStruct((B,), jnp.int32),
    ],
)


def _region(v):
    q1 = (v >= Q).astype(jnp.int32)
    q2 = (v >= 2 * Q).astype(jnp.int32)
    q3 = (v >= 3 * Q).astype(jnp.int32)
    return q1 + q2 + q3


def _sc_body(idx_hbm, c_hbm, nz_hbm, out_hbm,
             idx_v, idx2_v, c_v, nz_v, out_v, sems, nsem):
    wid = lax.axis_index("s") * NC + lax.axis_index("c")

    pltpu.sync_copy(idx_hbm.at[pl.ds(wid * BPW, BPW)], idx_v)
    nz_copy = pltpu.async_copy(
        nz_hbm.at[pl.ds((wid % 8) * BPW, BPW),
                  pl.ds((wid // 8) * D, D)], nz_v, nsem)

    @plsc.parallel_loop(0, BPW, step=L)
    def _mkidx(i):
        v = idx_v[pl.ds(i, L)]
        q = _region(v)
        y = v - q * Q
        idx2_v[pl.ds(i, L)] = 2 * y + lax.shift_right_logical(q, 1)

    def fire(c):
        par = c % 2
        return pltpu.async_copy(
            c_hbm.at[idx2_v.at[pl.ds(c * CHUNK, CHUNK)]],
            c_v.at[pl.ds(par * CHUNK, CHUNK), :],
            sems[par])

    lane = lax.iota(jnp.int32, L)
    pending = {0: fire(0), 1: fire(1)}
    nz_copy.wait()

    for c in range(NCHUNK):
        pending.pop(c).wait()
        par = c % 2

        @plsc.parallel_loop(0, CHUNK, step=1, unroll=4)
        def _row(j, _c=c, _par=par):
            jg = _c * CHUNK + j
            gi = plsc.load_gather(idx_v, [jnp.full((L,), jg, jnp.int32)])
            base = (_region(gi) & 1) * 32 + lane
            row16 = jnp.full((L,), _par * CHUNK + j, jnp.int32)
            locp = plsc.load_gather(c_v, [row16, base])
            sclp = plsc.load_gather(c_v, [row16, base + L])
            la, lb = plsc.unpack(
                plsc.bitcast(locp, jnp.bfloat16),
                format=plsc.PackFormat.INTERLEAVED,
                preferred_element_type=jnp.float32)
            sa, sb = plsc.unpack(
                plsc.bitcast(sclp, jnp.bfloat16),
                format=plsc.PackFormat.INTERLEAVED,
                preferred_element_type=jnp.float32)
            j16 = jnp.full((L,), j, jnp.int32)
            chunk16 = jnp.full((L,), _c, jnp.int32)
            i16 = lax.shift_right_logical(lane, 3)
            s16 = lane & 7
            plsc.store_scatter(
                out_v, [i16, chunk16, s16, j16],
                la + sa * nz_v[jg, pl.ds(0, L)])
            plsc.store_scatter(
                out_v, [i16 + 2, chunk16, s16, j16],
                lb + sb * nz_v[jg, pl.ds(L, L)])

        if c + 2 < NCHUNK:
            pending[c + 2] = fire(c + 2)

    pltpu.sync_copy(out_v, out_hbm.at[:, pl.ds(NCHUNK * wid, NCHUNK), :, :])


@functools.partial(
    pl.kernel,
    out_type=jax.ShapeDtypeStruct((4, B // CHUNK, 8, CHUNK), jnp.float32),
    mesh=plsc.VectorSubcoreMesh(core_axis_name="c", subcore_axis_name="s"),
    compiler_params=pltpu.CompilerParams(
        use_tc_tiling_on_sc=False, needs_layout_passes=False),
    scratch_types=[
        pltpu.VMEM((BPW,), jnp.int32),
        pltpu.VMEM((BPW,), jnp.int32),
        pltpu.VMEM((2 * CHUNK, 64), jnp.float32),
        pltpu.VMEM((BPW, D), jnp.float32),
        pltpu.VMEM((4, NCHUNK, 8, CHUNK), jnp.float32),
        [pltpu.SemaphoreType.DMA, pltpu.SemaphoreType.DMA],
        pltpu.SemaphoreType.DMA,
    ],
)
def _sc_kernel(idx_hbm, c_hbm, nz_hbm, out_hbm,
               idx_v, idx2_v, c_v, nz_v, out_v, sems, nsem):
    _sc_body(idx_hbm, c_hbm, nz_hbm, out_hbm,
             idx_v, idx2_v, c_v, nz_v, out_v, sems, nsem)


def kernel(inputs, untransformed_loc, untransformed_scale, noise):
    ul_t = untransformed_loc.T
    us_t = untransformed_scale.T
    comb = _tc_prep(ul_t, us_t, ul_t, us_t,
                    ul_t, us_t, ul_t, us_t).reshape(RROWS, 64)
    nz_t = noise.T
    nz2, idx = _nz_prep(nz_t, nz_t, nz_t, nz_t, inputs.T)
    out4 = _sc_kernel(idx, comb, nz2)
    return jnp.transpose(out4, (1, 3, 0, 2)).reshape(B, D)


# speedup vs baseline: 1.0088x; 1.0088x over previous
import functools

import jax
import jax.numpy as jnp
from jax import lax
from jax.experimental import pallas as pl
from jax.experimental.pallas import tpu as pltpu
from jax.experimental.pallas import tpu_sc as plsc

G = 100000
D = 32
B = 16384
EPS = 1e-7

NC = 2
NS = 16
L = 16
NW = NC * NS
BPW = B // NW
CHUNK = 128
NCHUNK = BPW // CHUNK

Q = 25600
RROWS = 2 * Q

BLKC = 5120
TCGRID = Q // BLKC


def _loc_of(x):
    return EPS + 1.0 / (1.0 + jnp.exp(-x))


def _scale_of(x):
    return EPS + jnp.maximum(x, 0.0) + jnp.log(1.0 + jnp.exp(-jnp.abs(x)))


def _pack16(x):
    b = lax.bitcast_convert_type(
        x.astype(jnp.bfloat16), jnp.uint16).astype(jnp.uint32)
    w = b[0:16, :] | (b[16:32, :] << 16)
    return lax.bitcast_convert_type(w, jnp.float32)


def _tc_prep_body(ul0, us0, ul1, us1, ul2, us2, ul3, us3, c_ref):
    srcs = []
    for ulr, usr in ((ul0, us0), (ul1, us1), (ul2, us2), (ul3, us3)):
        srcs.append(_pack16(_loc_of(ulr[...])))
        srcs.append(_pack16(_scale_of(usr[...])))
    for c in range(BLKC // 128):
        lo, hi = c * 128, (c + 1) * 128
        sq = jnp.concatenate([s[:, lo:hi] for s in srcs], axis=0)
        c_ref[pl.ds(c * 128 * 128, 128 * 128)] = sq.T.reshape(128 * 128)


def _in_spec(r):
    return pl.BlockSpec((32, BLKC), lambda i, _r=r: (0, i + _r * TCGRID))


_tc_prep = pl.pallas_call(
    _tc_prep_body,
    grid=(TCGRID,),
    in_specs=[_in_spec(r) for r in (0, 0, 1, 1, 2, 2, 3, 3)],
    out_specs=pl.BlockSpec((BLKC * 128,), lambda i: (i,)),
    out_shape=jax.ShapeDtypeStruct((RROWS * 64,), jnp.float32),
)


NZBLK = 2048


def _nz_prep_body(n0, n1, n2, n3, it_ref, nz_ref, idx_ref):
    qs = [n0[...], n1[...], n2[...], n3[...]]
    for c in range(NZBLK // 128):
        lo, hi = c * 128, (c + 1) * 128
        sq = jnp.concatenate([q[:, lo:hi] for q in qs], axis=0)
        nz_ref[lo:hi, :] = sq.T
    idx_ref[...] = it_ref[...].reshape(B // 2)


_nz_prep = pl.pallas_call(
    _nz_prep_body,
    grid=(2,),
    in_specs=[
        pl.BlockSpec((32, NZBLK), lambda i: (0, i)),
        pl.BlockSpec((32, NZBLK), lambda i: (0, i + 2)),
        pl.BlockSpec((32, NZBLK), lambda i: (0, i + 4)),
        pl.BlockSpec((32, NZBLK), lambda i: (0, i + 6)),
        pl.BlockSpec((1, B // 2), lambda i: (0, i)),
    ],
    out_specs=[
        pl.BlockSpec((NZBLK, 128), lambda i: (i, 0)),
        pl.BlockSpec((B // 2,), lambda i: (i,)),
    ],
    out_shape=[
        jax.ShapeDtypeStruct((B // 4, 128), jnp.float32),
        jax.ShapeDtypeStruct((B,), jnp.int32),
    ],
)


def _region(v):
    q1 = (v >= Q).astype(jnp.int32)
    q2 = (v >= 2 * Q).astype(jnp.int32)
    q3 = (v >= 3 * Q).astype(jnp.int32)
    return q1 + q2 + q3


def _sc_body(idx_hbm, c_hbm, nz_hbm, out_hbm,
             idx_v, idx2_v, c_v, nz_v, out_v, sems, nsem):
    wid = lax.axis_index("s") * NC + lax.axis_index("c")

    pltpu.sync_copy(idx_hbm.at[pl.ds(wid * BPW, BPW)], idx_v)
    nz_copy = pltpu.async_copy(
        nz_hbm.at[pl.ds((wid % 8) * BPW, BPW),
                  pl.ds((wid // 8) * D, D)], nz_v, nsem)

    @plsc.parallel_loop(0, BPW, step=L)
    def _mkidx(i):
        v = idx_v[pl.ds(i, L)]
        q = _region(v)
        y = v - q * Q
        idx2_v[pl.ds(i, L)] = 2 * y + lax.shift_right_logical(q, 1)

    def fire(c):
        par = c % 2
        return pltpu.async_copy(
            c_hbm.at[idx2_v.at[pl.ds(c * CHUNK, CHUNK)]],
            c_v.at[pl.ds(par * CHUNK, CHUNK), :],
            sems[par])

    lane = lax.iota(jnp.int32, L)
    pending = {0: fire(0), 1: fire(1)}
    nz_copy.wait()

    for c in range(NCHUNK):
        pending.pop(c).wait()
        par = c % 2

        @plsc.parallel_loop(0, CHUNK, step=1, unroll=2)
        def _row(j, _c=c, _par=par):
            jg = _c * CHUNK + j
            gi = plsc.load_gather(idx_v, [jnp.full((L,), jg, jnp.int32)])
            base = (_region(gi) & 1) * 32 + lane
            row16 = jnp.full((L,), _par * CHUNK + j, jnp.int32)
            locp = plsc.load_gather(c_v, [row16, base])
            sclp = plsc.load_gather(c_v, [row16, base + L])
            la, lb = plsc.unpack(
                plsc.bitcast(locp, jnp.bfloat16),
                format=plsc.PackFormat.INTERLEAVED,
                preferred_element_type=jnp.float32)
            sa, sb = plsc.unpack(
                plsc.bitcast(sclp, jnp.bfloat16),
                format=plsc.PackFormat.INTERLEAVED,
                preferred_element_type=jnp.float32)
            j16 = jnp.full((L,), j, jnp.int32)
            chunk16 = jnp.full((L,), _c, jnp.int32)
            i16 = lax.shift_right_logical(lane, 3)
            s16 = lane & 7
            plsc.store_scatter(
                out_v, [i16, chunk16, s16, j16],
                la + sa * nz_v[jg, pl.ds(0, L)])
            plsc.store_scatter(
                out_v, [i16 + 2, chunk16, s16, j16],
                lb + sb * nz_v[jg, pl.ds(L, L)])

        if c + 2 < NCHUNK:
            pending[c + 2] = fire(c + 2)

    pltpu.sync_copy(out_v, out_hbm.at[:, pl.ds(NCHUNK * wid, NCHUNK), :, :])


@functools.partial(
    pl.kernel,
    out_type=jax.ShapeDtypeStruct((4, B // CHUNK, 8, CHUNK), jnp.float32),
    mesh=plsc.VectorSubcoreMesh(core_axis_name="c", subcore_axis_name="s"),
    compiler_params=pltpu.CompilerParams(
        use_tc_tiling_on_sc=False, needs_layout_passes=False),
    scratch_types=[
        pltpu.VMEM((BPW,), jnp.int32),
        pltpu.VMEM((BPW,), jnp.int32),
        pltpu.VMEM((2 * CHUNK, 64), jnp.float32),
        pltpu.VMEM((BPW, D), jnp.float32),
        pltpu.VMEM((4, NCHUNK, 8, CHUNK), jnp.float32),
        [pltpu.SemaphoreType.DMA, pltpu.SemaphoreType.DMA],
        pltpu.SemaphoreType.DMA,
    ],
)
def _sc_kernel(idx_hbm, c_hbm, nz_hbm, out_hbm,
               idx_v, idx2_v, c_v, nz_v, out_v, sems, nsem):
    _sc_body(idx_hbm, c_hbm, nz_hbm, out_hbm,
             idx_v, idx2_v, c_v, nz_v, out_v, sems, nsem)


def kernel(inputs, untransformed_loc, untransformed_scale, noise):
    ul_t = untransformed_loc.T
    us_t = untransformed_scale.T
    comb = _tc_prep(ul_t, us_t, ul_t, us_t,
                    ul_t, us_t, ul_t, us_t).reshape(RROWS, 64)
    nz_t = noise.T
    nz2, idx = _nz_prep(nz_t, nz_t, nz_t, nz_t, inputs.T)
    out4 = _sc_kernel(idx, comb, nz2)
    return jnp.transpose(out4, (1, 3, 0, 2)).reshape(B, D)
